# baseline (speedup 1.0000x reference)
import jax
import jax.numpy as jnp
from jax import lax
from jax.experimental import pallas as pl
from jax.experimental.pallas import tpu as pltpu
from jax.experimental.pallas import tpu_sc as plsc

N = 10000
D = 128
K = 128
E = 160000

NC = 2
NS = 16
NW = NC * NS
CB = 80
NCHUNK = 63
EPAD = NW * NCHUNK * CB
NP = 10240
RPT = NP // NS
ZCH = 80
NZ = RPT // ZCH

NBUF = 3

BLK = 2000
GRID = N // BLK

_SELU_SCALE = 1.0507009873554805
_SELU_ALPHA = 1.6732632423543772


def _make_sc_body(grow, srow):
    def body(table, cei, zeros, out, gidx_v, sidx_v, rows_v, agg_sh, sem, zsem):
        c = lax.axis_index("c")
        s = lax.axis_index("s")
        w = c * NS + s

        pltpu.sync_copy(cei.at[grow, w], gidx_v)
        pltpu.sync_copy(cei.at[srow, w], sidx_v)

        pltpu.async_copy(table.at[gidx_v.at[0]], rows_v.at[1], sem)
        zbuf = rows_v.at[0, pl.ds(0, ZCH)]
        pltpu.sync_copy(zeros, zbuf)
        rbase = s * RPT
        zd = [pltpu.async_copy(zbuf, agg_sh.at[pl.ds(rbase + j * ZCH, ZCH)], zsem)
              for j in range(NZ)]
        for d in zd:
            d.wait()
        for p in range(1, NBUF):
            pltpu.async_copy(table.at[gidx_v.at[p]], rows_v.at[(p + 1) % NBUF], sem)
        plsc.subcore_barrier()

        def chunk(j, carry):
            buf = rows_v.at[lax.rem(j + 1, NBUF)]
            pltpu.make_async_copy(table.at[gidx_v.at[0]], buf, sem).wait()
            pltpu.sync_copy(buf, agg_sh.at[sidx_v.at[j]], add=True)

            @pl.when(j + NBUF < NCHUNK)
            def _next():
                jn = j + NBUF
                pltpu.async_copy(table.at[gidx_v.at[jn]], rows_v.at[lax.rem(jn + 1, NBUF)], sem)

            return carry

        lax.fori_loop(0, NCHUNK, chunk, 0)
        plsc.subcore_barrier()

        wd = []
        for j in range(NZ):
            b = j % NBUF
            if j >= NBUF:
                wd[j - NBUF].wait()
            bb = rows_v.at[b, pl.ds(0, ZCH)]
            pltpu.sync_copy(agg_sh.at[pl.ds(rbase + j * ZCH, ZCH)], bb)
            wd.append(pltpu.async_copy(bb, out.at[c, pl.ds(rbase + j * ZCH, ZCH)], zsem))
        for d in wd[-NBUF:]:
            d.wait()

    return body


_SC_MESH = plsc.VectorSubcoreMesh(
    core_axis_name="c", subcore_axis_name="s", num_cores=NC, num_subcores=NS)


def _sc_scatter(table, cei, grow, srow):
    zeros = jnp.zeros((ZCH, D), jnp.float32)
    return pl.kernel(
        _make_sc_body(grow, srow),
        out_type=jax.ShapeDtypeStruct((NC, NP, D), jnp.float32),
        mesh=_SC_MESH,
        scratch_types=[
            pltpu.VMEM((NCHUNK, CB), jnp.int32),
            pltpu.VMEM((NCHUNK, CB), jnp.int32),
            pltpu.VMEM((NBUF, CB, D), jnp.float32),
            pltpu.VMEM_SHARED((NP, D), jnp.float32),
            pltpu.SemaphoreType.DMA,
            pltpu.SemaphoreType.DMA,
        ],
        name="sc_edge_scatter",
    )(table, cei, zeros)


_F32 = jnp.float32


def _row_spec():
    return pl.BlockSpec((BLK, D), lambda i: (i, 0))


def _part_specs():
    return [pl.BlockSpec((1, BLK, D), lambda i: (0, i, 0)),
            pl.BlockSpec((1, BLK, D), lambda i: (1, i, 0))]


def _full_spec():
    return pl.BlockSpec((D, D), lambda i: (0, 0))


def _vec_spec():
    return pl.BlockSpec((1, D), lambda i: (0, 0))


def _tc_mm_body(v_ref, w_ref, o_ref):
    o_ref[...] = jnp.dot(v_ref[...], w_ref[...], preferred_element_type=_F32)


def _tc_mm(v, w):
    return pl.pallas_call(
        _tc_mm_body,
        grid=(GRID,),
        in_specs=[_row_spec(), _full_spec()],
        out_specs=_row_spec(),
        out_shape=jax.ShapeDtypeStruct((N, D), _F32),
    )(v, w)


def _tc_conv1_body(parts0, parts1, xr_ref, wrel_ref, brel_ref, h1_ref):
    agg = parts0[0] + parts1[0]
    h1 = (jnp.dot(agg, wrel_ref[...], preferred_element_type=_F32)
          + brel_ref[...] + xr_ref[...])
    h1_ref[...] = jnp.maximum(h1, 0.0)


def _tc_conv1(parts, xr, w_rel, b_rel):
    return pl.pallas_call(
        _tc_conv1_body,
        grid=(GRID,),
        in_specs=_part_specs() + [_row_spec(), _full_spec(), _vec_spec()],
        out_specs=_row_spec(),
        out_shape=jax.ShapeDtypeStruct((N, D), _F32),
    )(parts, parts, xr, w_rel, b_rel)


def _tc_conv2_body(parts0, parts1, h1r_ref, wrel_ref, brel_ref,
                   w1_ref, b1_ref, w2_ref, b2_ref,
                   h_ref, s_ref, n2c_ref):
    agg = parts0[0] + parts1[0]
    h = (jnp.dot(agg, wrel_ref[...], preferred_element_type=_F32)
         + brel_ref[...] + h1r_ref[...])
    h_ref[...] = h
    l1 = jnp.dot(h, w1_ref[...], preferred_element_type=_F32) + b1_ref[...]
    lg = jnp.dot(l1, w2_ref[...], preferred_element_type=_F32) + b2_ref[...]
    mx = jnp.max(lg, axis=1, keepdims=True)
    ex = jnp.exp(lg - mx)
    sm = ex / jnp.sum(ex, axis=1, keepdims=True)
    s_ref[...] = sm
    iota = lax.broadcasted_iota(jnp.int32, (BLK, K), 1)
    mxs = jnp.max(sm, axis=1, keepdims=True)
    n2c = jnp.min(jnp.where(sm == mxs, iota, K), axis=1)
    n2c_ref[...] = jnp.broadcast_to(n2c[:, None], (BLK, 8))


def _tc_conv2(parts, h1r, w_rel, b_rel, w1, b1, w2, b2):
    return pl.pallas_call(
        _tc_conv2_body,
        grid=(GRID,),
        in_specs=_part_specs() + [_row_spec(), _full_spec(), _vec_spec(),
                                  _full_spec(), _vec_spec(), _full_spec(), _vec_spec()],
        out_specs=[
            _row_spec(),
            _row_spec(),
            pl.BlockSpec((BLK, 8), lambda i: (i, 0)),
        ],
        out_shape=[
            jax.ShapeDtypeStruct((N, D), _F32),
            jax.ShapeDtypeStruct((N, K), _F32),
            jax.ShapeDtypeStruct((N, 8), jnp.int32),
        ],
    )(parts, parts, h1r, w_rel, b_rel, w1, b1, w2, b2)


def _tc_contract_body(s_ref, h_ref, ss_ref, sth_ref, cs_ref):
    i = pl.program_id(0)
    sm = s_ref[...]
    ssb = lax.dot_general(sm, sm, (((0,), (0,)), ((), ())),
                          preferred_element_type=_F32)
    sthb = lax.dot_general(sm, h_ref[...], (((0,), (0,)), ((), ())),
                           preferred_element_type=_F32)
    csb = jnp.sum(sm, axis=0, keepdims=True)

    @pl.when(i == 0)
    def _init():
        ss_ref[...] = ssb
        sth_ref[...] = sthb
        cs_ref[...] = csb

    @pl.when(i > 0)
    def _acc():
        ss_ref[...] += ssb
        sth_ref[...] += sthb
        cs_ref[...] += csb


def _tc_contract(s, h):
    return pl.pallas_call(
        _tc_contract_body,
        grid=(GRID,),
        in_specs=[_row_spec(), _row_spec()],
        out_specs=[
            pl.BlockSpec((K, K), lambda i: (0, 0)),
            pl.BlockSpec((K, D), lambda i: (0, 0)),
            pl.BlockSpec((1, K), lambda i: (0, 0)),
        ],
        out_shape=[
            jax.ShapeDtypeStruct((K, K), _F32),
            jax.ShapeDtypeStruct((K, D), _F32),
            jax.ShapeDtypeStruct((1, K), _F32),
        ],
    )(s, h)


def _tc3_body(t0_ref, t1_ref, s_ref, ss_ref, sth_ref, cs_ref,
              emb_ref, closs_ref, oadj_acc, ca_acc, m_acc):
    i = pl.program_id(0)

    @pl.when(i == 0)
    def _init():
        oadj_acc[...] = jnp.zeros_like(oadj_acc)
        ca_acc[...] = jnp.zeros_like(ca_acc)
        m_acc[0] = 0.0

    t = t0_ref[0] + t1_ref[0]
    sm = s_ref[...]
    deg = jnp.sum(t, axis=1, keepdims=True)
    oadj_acc[...] += lax.dot_general(sm, t, (((0,), (0,)), ((), ())),
                                     preferred_element_type=_F32)
    ca_acc[...] += jnp.sum(sm * deg, axis=0, keepdims=True)
    m_acc[0] += jnp.sum(deg)

    @pl.when(i == GRID - 1)
    def _fin():
        oadj = oadj_acc[...]
        ca = ca_acc[...]
        m = m_acc[0] * 0.5
        eye = (lax.broadcasted_iota(jnp.int32, (K, K), 0)
               == lax.broadcasted_iota(jnp.int32, (K, K), 1)).astype(_F32)
        tr_out = jnp.sum(oadj * eye)
        tr_norm = jnp.sum(ca * ca) / (2.0 * m)
        spectral = -(tr_out - tr_norm) / (2.0 * m)
        ss = ss_ref[...]
        ssn = jnp.sqrt(jnp.sum(ss * ss))
        dmat = ss / ssn - eye / jnp.sqrt(jnp.float32(K))
        ortho = jnp.sqrt(jnp.sum(dmat * dmat))
        cs = cs_ref[...]
        cluster = jnp.sqrt(jnp.sum(cs * cs)) / N * jnp.sqrt(jnp.float32(K)) - 1.0
        closs_ref[...] = (spectral + ortho + cluster)[None, None]
        sth = sth_ref[...]
        emb_ref[...] = _SELU_SCALE * jnp.where(
            sth > 0, sth, _SELU_ALPHA * (jnp.exp(sth) - 1.0))


def _tc3(tparts, s, ss, sth, cs):
    return pl.pallas_call(
        _tc3_body,
        grid=(GRID,),
        in_specs=_part_specs() + [
            _row_spec(),
            pl.BlockSpec((K, K), lambda i: (0, 0)),
            pl.BlockSpec((K, D), lambda i: (0, 0)),
            pl.BlockSpec((1, K), lambda i: (0, 0)),
        ],
        out_specs=[
            pl.BlockSpec((K, D), lambda i: (0, 0)),
            pl.BlockSpec((1, 1), lambda i: (0, 0)),
        ],
        out_shape=[
            jax.ShapeDtypeStruct((K, D), _F32),
            jax.ShapeDtypeStruct((1, 1), _F32),
        ],
        scratch_shapes=[
            pltpu.VMEM((K, K), _F32),
            pltpu.VMEM((1, K), _F32),
            pltpu.SMEM((1,), _F32),
        ],
    )(tparts, tparts, s, ss, sth, cs)


def kernel(x, hyper_edge_index, c_edge_index, W_rel, b_rel, W_root, W1, b1, W2, b2):
    del hyper_edge_index
    src, dst = c_edge_index[0], c_edge_index[1]
    npad = EPAD - E
    if npad:
        zpad = jnp.zeros((npad,), jnp.int32)
        dpad = N + (jnp.arange(npad, dtype=jnp.int32) % (NP - N))
        cei = jnp.stack([
            jnp.concatenate([src, zpad]),
            jnp.concatenate([dst, dpad]),
            jnp.concatenate([dst, zpad]),
            jnp.concatenate([src, dpad]),
        ]).reshape(4, NW, NCHUNK, CB)
    else:
        cei = jnp.stack([src, dst, dst, src]).reshape(4, NW, NCHUNK, CB)
    b_rel2 = b_rel.reshape(1, D)
    b12 = b1.reshape(1, D)
    b22 = b2.reshape(1, K)

    agg1 = _sc_scatter(x, cei, 0, 1)
    xr = _tc_mm(x, W_root)
    h1 = _tc_conv1(agg1, xr, W_rel, b_rel2)
    agg2 = _sc_scatter(h1, cei, 0, 1)
    h1r = _tc_mm(h1, W_root)
    h, s, n2c8 = _tc_conv2(agg2, h1r, W_rel, b_rel2, W1, b12, W2, b22)
    t = _sc_scatter(s, cei, 2, 3)
    ss, sth, cs = _tc_contract(s, h)
    emb, closs = _tc3(t, s, ss, sth, cs)
    return h, n2c8[:, 0], emb, closs[0, 0]


# speedup vs baseline: 1.8497x; 1.8497x over previous
import jax
import jax.numpy as jnp
from jax import lax
from jax.experimental import pallas as pl
from jax.experimental.pallas import tpu as pltpu
from jax.experimental.pallas import tpu_sc as plsc

N = 10000
D = 128
K = 128
E = 160000

NC = 2
NS = 16
NW = NC * NS
CB = 125
NCHUNK = 40
EPAD = NW * NCHUNK * CB
NP = 10240
RPT = NP // NS
ZCH = 80
NZ = RPT // ZCH

NBUF = 2

BLK = 2000
GRID = N // BLK

_SELU_SCALE = 1.0507009873554805
_SELU_ALPHA = 1.6732632423543772


def _make_sc_body(grow, srow):
    def body(table, cei, zeros, out, gidx_v, sidx_v, rows_v, agg_sh, sem, zsem):
        c = lax.axis_index("c")
        s = lax.axis_index("s")
        w = c * NS + s

        pltpu.sync_copy(cei.at[grow, w], gidx_v)
        pltpu.sync_copy(cei.at[srow, w], sidx_v)

        pltpu.async_copy(table.at[gidx_v.at[0]], rows_v.at[1], sem)
        zbuf = rows_v.at[0, pl.ds(0, ZCH)]
        pltpu.sync_copy(zeros, zbuf)
        rbase = s * RPT
        zd = [pltpu.async_copy(zbuf, agg_sh.at[pl.ds(rbase + j * ZCH, ZCH)], zsem)
              for j in range(NZ)]
        for d in zd:
            d.wait()
        for p in range(1, NBUF):
            pltpu.async_copy(table.at[gidx_v.at[p]], rows_v.at[(p + 1) % NBUF], sem)
        plsc.subcore_barrier()

        def chunk(j, carry):
            buf = rows_v.at[lax.rem(j + 1, NBUF)]
            pltpu.make_async_copy(table.at[gidx_v.at[0]], buf, sem).wait()
            pltpu.sync_copy(buf, agg_sh.at[sidx_v.at[j]], add=True)

            @pl.when(j + NBUF < NCHUNK)
            def _next():
                jn = j + NBUF
                pltpu.async_copy(table.at[gidx_v.at[jn]], rows_v.at[lax.rem(jn + 1, NBUF)], sem)

            return carry

        lax.fori_loop(0, NCHUNK, chunk, 0)
        plsc.subcore_barrier()

        wd = []
        for j in range(NZ):
            b = j % NBUF
            if j >= NBUF:
                wd[j - NBUF].wait()
            bb = rows_v.at[b, pl.ds(0, ZCH)]
            pltpu.sync_copy(agg_sh.at[pl.ds(rbase + j * ZCH, ZCH)], bb)
            wd.append(pltpu.async_copy(bb, out.at[c, pl.ds(rbase + j * ZCH, ZCH)], zsem))
        for d in wd[-NBUF:]:
            d.wait()

    return body


_SC_MESH = plsc.VectorSubcoreMesh(
    core_axis_name="c", subcore_axis_name="s", num_cores=NC, num_subcores=NS)


def _sc_scatter(table, cei, grow, srow):
    zeros = jnp.zeros((ZCH, D), jnp.float32)
    return pl.kernel(
        _make_sc_body(grow, srow),
        out_type=jax.ShapeDtypeStruct((NC, NP, D), jnp.float32),
        mesh=_SC_MESH,
        scratch_types=[
            pltpu.VMEM((NCHUNK, CB), jnp.int32),
            pltpu.VMEM((NCHUNK, CB), jnp.int32),
            pltpu.VMEM((NBUF, CB, D), jnp.float32),
            pltpu.VMEM_SHARED((NP, D), jnp.float32),
            pltpu.SemaphoreType.DMA,
            pltpu.SemaphoreType.DMA,
        ],
        name="sc_edge_scatter",
    )(table, cei, zeros)


_F32 = jnp.float32


def _row_spec():
    return pl.BlockSpec((BLK, D), lambda i: (i, 0))


def _part_specs():
    return [pl.BlockSpec((1, BLK, D), lambda i: (0, i, 0)),
            pl.BlockSpec((1, BLK, D), lambda i: (1, i, 0))]


def _full_spec():
    return pl.BlockSpec((D, D), lambda i: (0, 0))


def _vec_spec():
    return pl.BlockSpec((1, D), lambda i: (0, 0))


def _tc_mm_body(v_ref, w_ref, o_ref):
    o_ref[...] = jnp.dot(v_ref[...], w_ref[...], preferred_element_type=_F32)


def _tc_mm(v, w):
    return pl.pallas_call(
        _tc_mm_body,
        grid=(GRID,),
        in_specs=[_row_spec(), _full_spec()],
        out_specs=_row_spec(),
        out_shape=jax.ShapeDtypeStruct((N, D), _F32),
    )(v, w)


def _tc_conv1_body(parts0, parts1, xr_ref, wrel_ref, brel_ref, h1_ref):
    agg = parts0[0] + parts1[0]
    h1 = (jnp.dot(agg, wrel_ref[...], preferred_element_type=_F32)
          + brel_ref[...] + xr_ref[...])
    h1_ref[...] = jnp.maximum(h1, 0.0)


def _tc_conv1(parts, xr, w_rel, b_rel):
    return pl.pallas_call(
        _tc_conv1_body,
        grid=(GRID,),
        in_specs=_part_specs() + [_row_spec(), _full_spec(), _vec_spec()],
        out_specs=_row_spec(),
        out_shape=jax.ShapeDtypeStruct((N, D), _F32),
    )(parts, parts, xr, w_rel, b_rel)


def _tc_conv2_body(parts0, parts1, h1r_ref, wrel_ref, brel_ref,
                   w1_ref, b1_ref, w2_ref, b2_ref,
                   h_ref, s_ref, n2c_ref):
    agg = parts0[0] + parts1[0]
    h = (jnp.dot(agg, wrel_ref[...], preferred_element_type=_F32)
         + brel_ref[...] + h1r_ref[...])
    h_ref[...] = h
    l1 = jnp.dot(h, w1_ref[...], preferred_element_type=_F32) + b1_ref[...]
    lg = jnp.dot(l1, w2_ref[...], preferred_element_type=_F32) + b2_ref[...]
    mx = jnp.max(lg, axis=1, keepdims=True)
    ex = jnp.exp(lg - mx)
    sm = ex / jnp.sum(ex, axis=1, keepdims=True)
    s_ref[...] = sm
    iota = lax.broadcasted_iota(jnp.int32, (BLK, K), 1)
    mxs = jnp.max(sm, axis=1, keepdims=True)
    n2c = jnp.min(jnp.where(sm == mxs, iota, K), axis=1)
    n2c_ref[...] = jnp.broadcast_to(n2c[:, None], (BLK, 8))


def _tc_conv2(parts, h1r, w_rel, b_rel, w1, b1, w2, b2):
    return pl.pallas_call(
        _tc_conv2_body,
        grid=(GRID,),
        in_specs=_part_specs() + [_row_spec(), _full_spec(), _vec_spec(),
                                  _full_spec(), _vec_spec(), _full_spec(), _vec_spec()],
        out_specs=[
            _row_spec(),
            _row_spec(),
            pl.BlockSpec((BLK, 8), lambda i: (i, 0)),
        ],
        out_shape=[
            jax.ShapeDtypeStruct((N, D), _F32),
            jax.ShapeDtypeStruct((N, K), _F32),
            jax.ShapeDtypeStruct((N, 8), jnp.int32),
        ],
    )(parts, parts, h1r, w_rel, b_rel, w1, b1, w2, b2)


def _tc_contract_body(s_ref, h_ref, ss_ref, sth_ref, cs_ref):
    i = pl.program_id(0)
    sm = s_ref[...]
    ssb = lax.dot_general(sm, sm, (((0,), (0,)), ((), ())),
                          preferred_element_type=_F32)
    sthb = lax.dot_general(sm, h_ref[...], (((0,), (0,)), ((), ())),
                           preferred_element_type=_F32)
    csb = jnp.sum(sm, axis=0, keepdims=True)

    @pl.when(i == 0)
    def _init():
        ss_ref[...] = ssb
        sth_ref[...] = sthb
        cs_ref[...] = csb

    @pl.when(i > 0)
    def _acc():
        ss_ref[...] += ssb
        sth_ref[...] += sthb
        cs_ref[...] += csb


def _tc_contract(s, h):
    return pl.pallas_call(
        _tc_contract_body,
        grid=(GRID,),
        in_specs=[_row_spec(), _row_spec()],
        out_specs=[
            pl.BlockSpec((K, K), lambda i: (0, 0)),
            pl.BlockSpec((K, D), lambda i: (0, 0)),
            pl.BlockSpec((1, K), lambda i: (0, 0)),
        ],
        out_shape=[
            jax.ShapeDtypeStruct((K, K), _F32),
            jax.ShapeDtypeStruct((K, D), _F32),
            jax.ShapeDtypeStruct((1, K), _F32),
        ],
    )(s, h)


def _tc3_body(t0_ref, t1_ref, s_ref, ss_ref, sth_ref, cs_ref,
              emb_ref, closs_ref, oadj_acc, ca_acc, m_acc):
    i = pl.program_id(0)

    @pl.when(i == 0)
    def _init():
        oadj_acc[...] = jnp.zeros_like(oadj_acc)
        ca_acc[...] = jnp.zeros_like(ca_acc)
        m_acc[0] = 0.0

    t = t0_ref[0] + t1_ref[0]
    sm = s_ref[...]
    deg = jnp.sum(t, axis=1, keepdims=True)
    oadj_acc[...] += lax.dot_general(sm, t, (((0,), (0,)), ((), ())),
                                     preferred_element_type=_F32)
    ca_acc[...] += jnp.sum(sm * deg, axis=0, keepdims=True)
    m_acc[0] += jnp.sum(deg)

    @pl.when(i == GRID - 1)
    def _fin():
        oadj = oadj_acc[...]
        ca = ca_acc[...]
        m = m_acc[0] * 0.5
        eye = (lax.broadcasted_iota(jnp.int32, (K, K), 0)
               == lax.broadcasted_iota(jnp.int32, (K, K), 1)).astype(_F32)
        tr_out = jnp.sum(oadj * eye)
        tr_norm = jnp.sum(ca * ca) / (2.0 * m)
        spectral = -(tr_out - tr_norm) / (2.0 * m)
        ss = ss_ref[...]
        ssn = jnp.sqrt(jnp.sum(ss * ss))
        dmat = ss / ssn - eye / jnp.sqrt(jnp.float32(K))
        ortho = jnp.sqrt(jnp.sum(dmat * dmat))
        cs = cs_ref[...]
        cluster = jnp.sqrt(jnp.sum(cs * cs)) / N * jnp.sqrt(jnp.float32(K)) - 1.0
        closs_ref[...] = (spectral + ortho + cluster)[None, None]
        sth = sth_ref[...]
        emb_ref[...] = _SELU_SCALE * jnp.where(
            sth > 0, sth, _SELU_ALPHA * (jnp.exp(sth) - 1.0))


def _tc3(tparts, s, ss, sth, cs):
    return pl.pallas_call(
        _tc3_body,
        grid=(GRID,),
        in_specs=_part_specs() + [
            _row_spec(),
            pl.BlockSpec((K, K), lambda i: (0, 0)),
            pl.BlockSpec((K, D), lambda i: (0, 0)),
            pl.BlockSpec((1, K), lambda i: (0, 0)),
        ],
        out_specs=[
            pl.BlockSpec((K, D), lambda i: (0, 0)),
            pl.BlockSpec((1, 1), lambda i: (0, 0)),
        ],
        out_shape=[
            jax.ShapeDtypeStruct((K, D), _F32),
            jax.ShapeDtypeStruct((1, 1), _F32),
        ],
        scratch_shapes=[
            pltpu.VMEM((K, K), _F32),
            pltpu.VMEM((1, K), _F32),
            pltpu.SMEM((1,), _F32),
        ],
    )(tparts, tparts, s, ss, sth, cs)


def kernel(x, hyper_edge_index, c_edge_index, W_rel, b_rel, W_root, W1, b1, W2, b2):
    del hyper_edge_index
    cei = c_edge_index.reshape(2, NW, NCHUNK, CB)
    b_rel2 = b_rel.reshape(1, D)
    b12 = b1.reshape(1, D)
    b22 = b2.reshape(1, K)

    agg1 = _sc_scatter(x, cei, 0, 1)
    xr = _tc_mm(x, W_root)
    h1 = _tc_conv1(agg1, xr, W_rel, b_rel2)
    agg2 = _sc_scatter(h1, cei, 0, 1)
    h1r = _tc_mm(h1, W_root)
    h, s, n2c8 = _tc_conv2(agg2, h1r, W_rel, b_rel2, W1, b12, W2, b22)
    t = _sc_scatter(s, cei, 1, 0)
    ss, sth, cs = _tc_contract(s, h)
    emb, closs = _tc3(t, s, ss, sth, cs)
    return h, n2c8[:, 0], emb, closs[0, 0]


# speedup vs baseline: 1.8714x; 1.0117x over previous
import jax
import jax.numpy as jnp
from jax import lax
from jax.experimental import pallas as pl
from jax.experimental.pallas import tpu as pltpu
from jax.experimental.pallas import tpu_sc as plsc

N = 10000
D = 128
K = 128
E = 160000

NC = 2
NS = 16
NW = NC * NS
CB = 125
NCHUNK = 40
EPAD = NW * NCHUNK * CB
NP = 10240
RPT = NP // NS
ZCH = 80
NZ = RPT // ZCH

NBUF = 2

BLK = 5000
GRID = N // BLK

_SELU_SCALE = 1.0507009873554805
_SELU_ALPHA = 1.6732632423543772


def _make_sc_body(grow, srow):
    def body(table, cei, zeros, out, gidx_v, sidx_v, rows_v, agg_sh, sem, zsem):
        c = lax.axis_index("c")
        s = lax.axis_index("s")
        w = c * NS + s

        pltpu.sync_copy(cei.at[grow, w], gidx_v)
        pltpu.sync_copy(cei.at[srow, w], sidx_v)

        pltpu.async_copy(table.at[gidx_v.at[0]], rows_v.at[1], sem)
        zbuf = rows_v.at[0, pl.ds(0, ZCH)]
        pltpu.sync_copy(zeros, zbuf)
        rbase = s * RPT
        zd = [pltpu.async_copy(zbuf, agg_sh.at[pl.ds(rbase + j * ZCH, ZCH)], zsem)
              for j in range(NZ)]
        for d in zd:
            d.wait()
        for p in range(1, NBUF):
            pltpu.async_copy(table.at[gidx_v.at[p]], rows_v.at[(p + 1) % NBUF], sem)
        plsc.subcore_barrier()

        def chunk(j, carry):
            buf = rows_v.at[lax.rem(j + 1, NBUF)]
            pltpu.make_async_copy(table.at[gidx_v.at[0]], buf, sem).wait()
            pltpu.sync_copy(buf, agg_sh.at[sidx_v.at[j]], add=True)

            @pl.when(j + NBUF < NCHUNK)
            def _next():
                jn = j + NBUF
                pltpu.async_copy(table.at[gidx_v.at[jn]], rows_v.at[lax.rem(jn + 1, NBUF)], sem)

            return carry

        lax.fori_loop(0, NCHUNK, chunk, 0)
        plsc.subcore_barrier()

        wd = []
        for j in range(NZ):
            b = j % NBUF
            if j >= NBUF:
                wd[j - NBUF].wait()
            bb = rows_v.at[b, pl.ds(0, ZCH)]
            pltpu.sync_copy(agg_sh.at[pl.ds(rbase + j * ZCH, ZCH)], bb)
            wd.append(pltpu.async_copy(bb, out.at[c, pl.ds(rbase + j * ZCH, ZCH)], zsem))
        for d in wd[-NBUF:]:
            d.wait()

    return body


_SC_MESH = plsc.VectorSubcoreMesh(
    core_axis_name="c", subcore_axis_name="s", num_cores=NC, num_subcores=NS)


def _sc_scatter(table, cei, grow, srow):
    zeros = jnp.zeros((ZCH, D), jnp.float32)
    return pl.kernel(
        _make_sc_body(grow, srow),
        out_type=jax.ShapeDtypeStruct((NC, NP, D), jnp.float32),
        mesh=_SC_MESH,
        scratch_types=[
            pltpu.VMEM((NCHUNK, CB), jnp.int32),
            pltpu.VMEM((NCHUNK, CB), jnp.int32),
            pltpu.VMEM((NBUF, CB, D), jnp.float32),
            pltpu.VMEM_SHARED((NP, D), jnp.float32),
            pltpu.SemaphoreType.DMA,
            pltpu.SemaphoreType.DMA,
        ],
        name="sc_edge_scatter",
    )(table, cei, zeros)


_F32 = jnp.float32


def _row_spec():
    return pl.BlockSpec((BLK, D), lambda i: (i, 0))


def _part_specs():
    return [pl.BlockSpec((1, BLK, D), lambda i: (0, i, 0)),
            pl.BlockSpec((1, BLK, D), lambda i: (1, i, 0))]


def _full_spec():
    return pl.BlockSpec((D, D), lambda i: (0, 0))


def _vec_spec():
    return pl.BlockSpec((1, D), lambda i: (0, 0))


def _tc_mm_body(v_ref, w_ref, o_ref):
    o_ref[...] = jnp.dot(v_ref[...], w_ref[...], preferred_element_type=_F32)


def _tc_mm(v, w):
    return pl.pallas_call(
        _tc_mm_body,
        grid=(GRID,),
        in_specs=[_row_spec(), _full_spec()],
        out_specs=_row_spec(),
        out_shape=jax.ShapeDtypeStruct((N, D), _F32),
    )(v, w)


def _tc_conv1_body(parts0, parts1, xr_ref, wrel_ref, brel_ref, h1_ref):
    agg = parts0[0] + parts1[0]
    h1 = (jnp.dot(agg, wrel_ref[...], preferred_element_type=_F32)
          + brel_ref[...] + xr_ref[...])
    h1_ref[...] = jnp.maximum(h1, 0.0)


def _tc_conv1(parts, xr, w_rel, b_rel):
    return pl.pallas_call(
        _tc_conv1_body,
        grid=(GRID,),
        in_specs=_part_specs() + [_row_spec(), _full_spec(), _vec_spec()],
        out_specs=_row_spec(),
        out_shape=jax.ShapeDtypeStruct((N, D), _F32),
    )(parts, parts, xr, w_rel, b_rel)


def _tc_conv2_body(parts0, parts1, h1r_ref, wrel_ref, brel_ref,
                   w1_ref, b1_ref, w2_ref, b2_ref,
                   h_ref, s_ref, n2c_ref):
    agg = parts0[0] + parts1[0]
    h = (jnp.dot(agg, wrel_ref[...], preferred_element_type=_F32)
         + brel_ref[...] + h1r_ref[...])
    h_ref[...] = h
    l1 = jnp.dot(h, w1_ref[...], preferred_element_type=_F32) + b1_ref[...]
    lg = jnp.dot(l1, w2_ref[...], preferred_element_type=_F32) + b2_ref[...]
    mx = jnp.max(lg, axis=1, keepdims=True)
    ex = jnp.exp(lg - mx)
    sm = ex / jnp.sum(ex, axis=1, keepdims=True)
    s_ref[...] = sm
    iota = lax.broadcasted_iota(jnp.int32, (BLK, K), 1)
    mxs = jnp.max(sm, axis=1, keepdims=True)
    n2c = jnp.min(jnp.where(sm == mxs, iota, K), axis=1)
    n2c_ref[...] = jnp.broadcast_to(n2c[:, None], (BLK, 8))


def _tc_conv2(parts, h1r, w_rel, b_rel, w1, b1, w2, b2):
    return pl.pallas_call(
        _tc_conv2_body,
        grid=(GRID,),
        in_specs=_part_specs() + [_row_spec(), _full_spec(), _vec_spec(),
                                  _full_spec(), _vec_spec(), _full_spec(), _vec_spec()],
        out_specs=[
            _row_spec(),
            _row_spec(),
            pl.BlockSpec((BLK, 8), lambda i: (i, 0)),
        ],
        out_shape=[
            jax.ShapeDtypeStruct((N, D), _F32),
            jax.ShapeDtypeStruct((N, K), _F32),
            jax.ShapeDtypeStruct((N, 8), jnp.int32),
        ],
    )(parts, parts, h1r, w_rel, b_rel, w1, b1, w2, b2)


def _tc_contract_body(s_ref, h_ref, ss_ref, sth_ref, cs_ref):
    i = pl.program_id(0)
    sm = s_ref[...]
    ssb = lax.dot_general(sm, sm, (((0,), (0,)), ((), ())),
                          preferred_element_type=_F32)
    sthb = lax.dot_general(sm, h_ref[...], (((0,), (0,)), ((), ())),
                           preferred_element_type=_F32)
    csb = jnp.sum(sm, axis=0, keepdims=True)

    @pl.when(i == 0)
    def _init():
        ss_ref[...] = ssb
        sth_ref[...] = sthb
        cs_ref[...] = csb

    @pl.when(i > 0)
    def _acc():
        ss_ref[...] += ssb
        sth_ref[...] += sthb
        cs_ref[...] += csb


def _tc_contract(s, h):
    return pl.pallas_call(
        _tc_contract_body,
        grid=(GRID,),
        in_specs=[_row_spec(), _row_spec()],
        out_specs=[
            pl.BlockSpec((K, K), lambda i: (0, 0)),
            pl.BlockSpec((K, D), lambda i: (0, 0)),
            pl.BlockSpec((1, K), lambda i: (0, 0)),
        ],
        out_shape=[
            jax.ShapeDtypeStruct((K, K), _F32),
            jax.ShapeDtypeStruct((K, D), _F32),
            jax.ShapeDtypeStruct((1, K), _F32),
        ],
    )(s, h)


def _tc3_body(t0_ref, t1_ref, s_ref, ss_ref, sth_ref, cs_ref,
              emb_ref, closs_ref, oadj_acc, ca_acc, m_acc):
    i = pl.program_id(0)

    @pl.when(i == 0)
    def _init():
        oadj_acc[...] = jnp.zeros_like(oadj_acc)
        ca_acc[...] = jnp.zeros_like(ca_acc)
        m_acc[0] = 0.0

    t = t0_ref[0] + t1_ref[0]
    sm = s_ref[...]
    deg = jnp.sum(t, axis=1, keepdims=True)
    oadj_acc[...] += lax.dot_general(sm, t, (((0,), (0,)), ((), ())),
                                     preferred_element_type=_F32)
    ca_acc[...] += jnp.sum(sm * deg, axis=0, keepdims=True)
    m_acc[0] += jnp.sum(deg)

    @pl.when(i == GRID - 1)
    def _fin():
        oadj = oadj_acc[...]
        ca = ca_acc[...]
        m = m_acc[0] * 0.5
        eye = (lax.broadcasted_iota(jnp.int32, (K, K), 0)
               == lax.broadcasted_iota(jnp.int32, (K, K), 1)).astype(_F32)
        tr_out = jnp.sum(oadj * eye)
        tr_norm = jnp.sum(ca * ca) / (2.0 * m)
        spectral = -(tr_out - tr_norm) / (2.0 * m)
        ss = ss_ref[...]
        ssn = jnp.sqrt(jnp.sum(ss * ss))
        dmat = ss / ssn - eye / jnp.sqrt(jnp.float32(K))
        ortho = jnp.sqrt(jnp.sum(dmat * dmat))
        cs = cs_ref[...]
        cluster = jnp.sqrt(jnp.sum(cs * cs)) / N * jnp.sqrt(jnp.float32(K)) - 1.0
        closs_ref[...] = (spectral + ortho + cluster)[None, None]
        sth = sth_ref[...]
        emb_ref[...] = _SELU_SCALE * jnp.where(
            sth > 0, sth, _SELU_ALPHA * (jnp.exp(sth) - 1.0))


def _tc3(tparts, s, ss, sth, cs):
    return pl.pallas_call(
        _tc3_body,
        grid=(GRID,),
        in_specs=_part_specs() + [
            _row_spec(),
            pl.BlockSpec((K, K), lambda i: (0, 0)),
            pl.BlockSpec((K, D), lambda i: (0, 0)),
            pl.BlockSpec((1, K), lambda i: (0, 0)),
        ],
        out_specs=[
            pl.BlockSpec((K, D), lambda i: (0, 0)),
            pl.BlockSpec((1, 1), lambda i: (0, 0)),
        ],
        out_shape=[
            jax.ShapeDtypeStruct((K, D), _F32),
            jax.ShapeDtypeStruct((1, 1), _F32),
        ],
        scratch_shapes=[
            pltpu.VMEM((K, K), _F32),
            pltpu.VMEM((1, K), _F32),
            pltpu.SMEM((1,), _F32),
        ],
    )(tparts, tparts, s, ss, sth, cs)


def kernel(x, hyper_edge_index, c_edge_index, W_rel, b_rel, W_root, W1, b1, W2, b2):
    del hyper_edge_index
    cei = c_edge_index.reshape(2, NW, NCHUNK, CB)
    b_rel2 = b_rel.reshape(1, D)
    b12 = b1.reshape(1, D)
    b22 = b2.reshape(1, K)

    agg1 = _sc_scatter(x, cei, 0, 1)
    xr = _tc_mm(x, W_root)
    h1 = _tc_conv1(agg1, xr, W_rel, b_rel2)
    agg2 = _sc_scatter(h1, cei, 0, 1)
    h1r = _tc_mm(h1, W_root)
    h, s, n2c8 = _tc_conv2(agg2, h1r, W_rel, b_rel2, W1, b12, W2, b22)
    t = _sc_scatter(s, cei, 1, 0)
    ss, sth, cs = _tc_contract(s, h)
    emb, closs = _tc3(t, s, ss, sth, cs)
    return h, n2c8[:, 0], emb, closs[0, 0]
